# baseline (speedup 1.0000x reference)
import dataclasses
import functools

import jax
import jax.numpy as jnp
from jax import lax
from jax.experimental import pallas as pl
from jax.experimental.pallas import tpu as pltpu
from jax.experimental.pallas import tpu_sc as plsc

N = 10000
E = 160000
IN = 256
HID = 1024
OUT = 256
HALF = 5000
PADROWS = 5120
NPAD = 2 * PADROWS
DUMMY = 5100
EPAD = 163840
CH = 64
NCHUNK = 160
EROWS = EPAD // CH
DEGW = 16
STRIPE = 320
C1 = 1.0 / (1.0 + 1e-5) ** 0.5
SENTINEL = 1 << 30

@functools.cache
def _sc_compiler_params():
    cp = pltpu.CompilerParams()
    if "needs_layout_passes" in pltpu.CompilerParams.__dataclass_fields__:
        cp = dataclasses.replace(cp, needs_layout_passes=False)
    return cp


@functools.cache
def _mesh():
    return plsc.VectorSubcoreMesh(core_axis_name="c", subcore_axis_name="s")


def _fill2d(ref, rows, val):
    width = ref.shape[1]

    @pl.loop(0, rows)
    def _(i):
        for j in range(width // 16):
            ref[i, pl.ds(j * 16, 16)] = jnp.full((16,), val, ref.dtype)


def _compute_scatter_idx(rbuf, sidx, lo, nchunk):
    @pl.loop(0, nchunk)
    def _(ch):
        for j in range(CH // 16):
            r = rbuf[ch, pl.ds(j * 16, 16)]
            ok = (r >= lo) & (r < lo + HALF)
            sidx[ch, pl.ds(j * 16, 16)] = jnp.where(ok, r - lo, DUMMY)


@functools.cache
def _deg_kernel():
    return pl.kernel(
        _deg_body,
        out_type=jax.ShapeDtypeStruct((NPAD, DEGW), jnp.float32),
        mesh=_mesh(),
        scratch_types=[
            pltpu.VMEM((NCHUNK, CH), jnp.int32),
            pltpu.VMEM((NCHUNK, CH), jnp.int32),
            pltpu.VMEM((CH, DEGW), jnp.float32),
            pltpu.VMEM((STRIPE, DEGW), jnp.float32),
            pltpu.VMEM_SHARED((PADROWS, DEGW), jnp.float32),
        ],
    )


def _deg_body(rowp, deg_out, rbuf, sidx, ones, zstage, dacc):
    c = lax.axis_index("c")
    s = lax.axis_index("s")
    _fill2d(ones, CH, 1.0)
    _fill2d(zstage, STRIPE, 0.0)
    pltpu.sync_copy(zstage, dacc.at[pl.ds(s * STRIPE, STRIPE)])
    pltpu.sync_copy(rowp.at[pl.ds(s * NCHUNK, NCHUNK)], rbuf)
    _compute_scatter_idx(rbuf, sidx, c * HALF, NCHUNK)
    plsc.subcore_barrier()

    @pl.loop(0, NCHUNK)
    def _(ch):
        pltpu.sync_copy(ones, dacc.at[sidx.at[ch]], add=True)

    plsc.subcore_barrier()
    pltpu.sync_copy(dacc.at[pl.ds(s * STRIPE, STRIPE)],
                    deg_out.at[pl.ds(c * PADROWS + s * STRIPE, STRIPE)])


SPLIT = 2
SW = IN // SPLIT
RCH = 32
CHI = RCH * SPLIT
RNCH = EPAD // (16 * RCH)
E2ROWS = EPAD * SPLIT // CHI
ACC2 = PADROWS * SPLIT
G2 = NPAD * SPLIT
NBUF = 4
PCH = 64


@functools.cache
def _round_kernel():
    return pl.kernel(
        _round_body,
        out_type=jax.ShapeDtypeStruct((G2, SW), jnp.float32),
        mesh=_mesh(),
        scratch_types=[
            pltpu.VMEM((16,), jnp.int32),
            pltpu.VMEM((2 * NBUF, RCH), jnp.int32),
            pltpu.VMEM((2 * NBUF, CHI), jnp.int32),
            pltpu.VMEM((NBUF, RCH, SPLIT, SW), jnp.float32),
            pltpu.VMEM((PCH // SPLIT, DEGW), jnp.float32),
            pltpu.VMEM_SHARED((ACC2, SW), jnp.float32),
        ] + [pltpu.SemaphoreType.DMA] * (2 * NBUF),
        compiler_params=_sc_compiler_params(),
    )


def _round_body(g_in, cidx2, sidx2, dinv2, karr, g_out,
                kbuf, cbufg, ibufg, sbuf, dbuf, acc, *sems):
    gsem, ssem = sems[:NBUF], sems[NBUF:]
    c = lax.axis_index("c")
    s = lax.axis_index("s")
    zb = sbuf.at[0].reshape(CHI, SW)
    _fill2d(zb, CHI, 0.0)
    nz = (STRIPE * SPLIT) // CHI
    for k in range(nz):
        pltpu.sync_copy(zb,
                        acc.at[pl.ds(s * STRIPE * SPLIT + k * CHI, CHI)])
    pltpu.sync_copy(karr, kbuf)
    plsc.subcore_barrier()
    kval = lax.reduce_max(kbuf[pl.ds(0, 16)], axes=(0,))
    eblk = 2 * NBUF * RCH
    nblk_tot = EPAD // eblk
    lo = jnp.where(c == 0, 0, kval // eblk)
    hi = jnp.where(c == 0, (kval + eblk - 1) // eblk, nblk_tot)
    per = (hi - lo + 15) // 16
    bstart = lo + s * per
    bend = jnp.minimum(bstart + per, hi)

    @pl.loop(bstart, bend)
    def _(it):
        base = it * 2 * NBUF
        pltpu.sync_copy(cidx2.at[pl.ds(base, 2 * NBUF)], cbufg)
        pltpu.sync_copy(sidx2.at[pl.ds(c * E2ROWS + base, 2 * NBUF)], ibufg)
        hg = [pltpu.async_copy(g_in.at[cbufg.at[p]], sbuf.at[p], gsem[p])
              for p in range(NBUF)]
        hs = [None] * NBUF
        for p in range(NBUF):
            hg[p].wait()
            hs[p] = pltpu.async_copy(sbuf.at[p].reshape(CHI, SW),
                                     acc.at[ibufg.at[p]], ssem[p], add=True)
        for p in range(NBUF):
            hs[p].wait()
            hg[p] = pltpu.async_copy(g_in.at[cbufg.at[NBUF + p]], sbuf.at[p],
                                     gsem[p])
        for p in range(NBUF):
            hg[p].wait()
            hs[p] = pltpu.async_copy(sbuf.at[p].reshape(CHI, SW),
                                     acc.at[ibufg.at[NBUF + p]],
                                     ssem[p], add=True)
        for p in range(NBUF):
            hs[p].wait()

    plsc.subcore_barrier()
    npost = (STRIPE * SPLIT) // PCH
    for k in range(npost):
        rowbase = c * PADROWS + s * STRIPE + k * (PCH // SPLIT)
        pb = sbuf.at[0].reshape(PCH, SW)
        pltpu.sync_copy(acc.at[pl.ds(s * STRIPE * SPLIT + k * PCH, PCH)], pb)
        pltpu.sync_copy(dinv2.at[pl.ds(rowbase, PCH // SPLIT)], dbuf)

        @pl.loop(0, PCH // SPLIT)
        def _(i):
            dv = dbuf[i, pl.ds(0, 16)]
            for j in range(SPLIT):
                for l in range(SW // 16):
                    pb[i * SPLIT + j, pl.ds(l * 16, 16)] = (
                        pb[i * SPLIT + j, pl.ds(l * 16, 16)] * dv)

        pltpu.sync_copy(pb, g_out.at[pl.ds(rowbase * SPLIT, PCH)])


ROWBLK = 512
NBLK = NPAD // ROWBLK


def _prep_body(x_ref, deg_ref, g0_ref, dinv2_ref):
    xb = x_ref[...]
    db = deg_ref[...]
    fsum = jnp.sum(xb, axis=1, keepdims=True)
    finv = jnp.where(fsum != 0, 1.0 / fsum, 0.0)
    xn = xb * finv * 0.5
    d1 = db[:, 0:1]
    dinv = jnp.where(d1 > 0, lax.rsqrt(d1), 0.0)
    g0_ref[...] = xn * dinv
    dinv2_ref[...] = jnp.where(db > 0, 1.0 / db, 0.0)


_prep = pl.pallas_call(
    _prep_body,
    grid=(NBLK,),
    in_specs=[
        pl.BlockSpec((ROWBLK, IN), lambda i: (i, 0)),
        pl.BlockSpec((ROWBLK, DEGW), lambda i: (i, 0)),
    ],
    out_specs=[
        pl.BlockSpec((ROWBLK, IN), lambda i: (i, 0)),
        pl.BlockSpec((ROWBLK, DEGW), lambda i: (i, 0)),
    ],
    out_shape=[
        jax.ShapeDtypeStruct((NPAD, IN), jnp.float32),
        jax.ShapeDtypeStruct((NPAD, DEGW), jnp.float32),
    ],
)


def _mlp_body(x_ref, deg_ref, g1_ref, g2_ref, g3_ref, g4_ref,
              W1_ref, b1_ref, W2_ref, b2_ref,
              gm1_ref, bt1_ref, gm2_ref, bt2_ref, o_ref):
    xb = x_ref[...]
    fsum = jnp.sum(xb, axis=1, keepdims=True)
    finv = jnp.where(fsum != 0, 1.0 / fsum, 0.0)
    xn = xb * finv * 0.5
    sq = jnp.sqrt(deg_ref[:, 0:1])
    gacc = g1_ref[...] + g2_ref[...] + g3_ref[...] + g4_ref[...]
    y = (xn + sq * gacc) * 0.2
    a = y * (C1 * gm1_ref[...]) + bt1_ref[...]
    h = jnp.dot(a, W1_ref[...], preferred_element_type=jnp.float32) + b1_ref[...]
    h = jnp.maximum(h, 0.0)
    h = h * (C1 * gm2_ref[...]) + bt2_ref[...]
    o_ref[...] = jnp.dot(h, W2_ref[...],
                         preferred_element_type=jnp.float32) + b2_ref[...]


_mlp = pl.pallas_call(
    _mlp_body,
    grid=(NBLK,),
    in_specs=[
        pl.BlockSpec((ROWBLK, IN), lambda i: (i, 0)),
        pl.BlockSpec((ROWBLK, DEGW), lambda i: (i, 0)),
        pl.BlockSpec((ROWBLK, IN), lambda i: (i, 0)),
        pl.BlockSpec((ROWBLK, IN), lambda i: (i, 0)),
        pl.BlockSpec((ROWBLK, IN), lambda i: (i, 0)),
        pl.BlockSpec((ROWBLK, IN), lambda i: (i, 0)),
        pl.BlockSpec((IN, HID), lambda i: (0, 0)),
        pl.BlockSpec((1, HID), lambda i: (0, 0)),
        pl.BlockSpec((HID, OUT), lambda i: (0, 0)),
        pl.BlockSpec((1, OUT), lambda i: (0, 0)),
        pl.BlockSpec((1, IN), lambda i: (0, 0)),
        pl.BlockSpec((1, IN), lambda i: (0, 0)),
        pl.BlockSpec((1, HID), lambda i: (0, 0)),
        pl.BlockSpec((1, HID), lambda i: (0, 0)),
    ],
    out_specs=pl.BlockSpec((ROWBLK, OUT), lambda i: (i, 0)),
    out_shape=jax.ShapeDtypeStruct((NPAD, OUT), jnp.float32),
)


def kernel(x, edge_index, W1, b1, W2, b2, gamma1, beta1, gamma2, beta2):
    row = edge_index[0].astype(jnp.int32)
    col = edge_index[1].astype(jnp.int32)
    colr = col + jnp.where(col >= HALF, PADROWS - HALF, 0).astype(jnp.int32)
    rowflat = jnp.concatenate([row, jnp.full((EPAD - E,), SENTINEL, jnp.int32)])
    colflat = jnp.concatenate([colr, jnp.zeros((EPAD - E,), jnp.int32)])
    key = (rowflat >= HALF).astype(jnp.int32)
    c1s = jnp.cumsum(key)
    ksplit = (EPAD - c1s[-1]).astype(jnp.int32)
    pos = jnp.where(key == 0,
                    jnp.arange(EPAD, dtype=jnp.int32) - c1s,
                    ksplit + c1s - 1)
    rowflat = jnp.zeros((EPAD,), jnp.int32).at[pos].set(
        rowflat, unique_indices=True)
    colflat = jnp.zeros((EPAD,), jnp.int32).at[pos].set(
        colflat, unique_indices=True)
    karr = jnp.full((16,), ksplit, jnp.int32)
    rowp = rowflat.reshape(EROWS, CH)
    lanes = jnp.arange(SPLIT, dtype=jnp.int32)
    cidxr = colflat.reshape(EPAD // RCH, RCH)
    sidx_c = []
    for c in range(2):
        lo = c * HALF
        loc = jnp.where((rowflat >= lo) & (rowflat < lo + HALF),
                        rowflat - lo, DUMMY)
        sidx_c.append((loc[:, None] * SPLIT + lanes).reshape(E2ROWS, CHI))
    sidx2 = jnp.concatenate(sidx_c, axis=0)
    z = jnp.zeros((PADROWS - HALF, IN), jnp.float32)
    x_pad = jnp.concatenate([x[:HALF], z, x[HALF:], z], axis=0)

    deg = _deg_kernel()(rowp)
    g0, dinv2 = _prep(x_pad, deg)
    rnd = _round_kernel()
    r3 = lambda g: g.reshape(NPAD, SPLIT, SW)
    r256 = lambda g: g.reshape(NPAD, IN)
    g1 = rnd(r3(g0), cidxr, sidx2, dinv2, karr)
    g2 = rnd(r3(g1), cidxr, sidx2, dinv2, karr)
    g3 = rnd(r3(g2), cidxr, sidx2, dinv2, karr)
    g4 = rnd(r3(g3), cidxr, sidx2, dinv2, karr)
    g1, g2, g3, g4 = r256(g1), r256(g2), r256(g3), r256(g4)
    out_pad = _mlp(x_pad, deg, g1, g2, g3, g4,
                   W1, b1.reshape(1, HID), W2, b2.reshape(1, OUT),
                   gamma1.reshape(1, IN), beta1.reshape(1, IN),
                   gamma2.reshape(1, HID), beta2.reshape(1, HID))
    return jnp.concatenate([out_pad[:HALF], out_pad[PADROWS:PADROWS + HALF]],
                           axis=0)


# speedup vs baseline: 1.5467x; 1.5467x over previous
import dataclasses
import functools

import jax
import jax.numpy as jnp
from jax import lax
from jax.experimental import pallas as pl
from jax.experimental.pallas import tpu as pltpu
from jax.experimental.pallas import tpu_sc as plsc

N = 10000
E = 160000
IN = 256
HID = 1024
OUT = 256
HALF = 5000
PADROWS = 5120
NPAD = 2 * PADROWS
DUMMY = 5100
EPAD = 163840
CH = 64
NCHUNK = 160
EROWS = EPAD // CH
DEGW = 16
STRIPE = 320
C1 = 1.0 / (1.0 + 1e-5) ** 0.5
SENTINEL = 1 << 30

@functools.cache
def _sc_compiler_params():
    cp = pltpu.CompilerParams()
    if "needs_layout_passes" in pltpu.CompilerParams.__dataclass_fields__:
        cp = dataclasses.replace(cp, needs_layout_passes=False)
    return cp


@functools.cache
def _mesh():
    return plsc.VectorSubcoreMesh(core_axis_name="c", subcore_axis_name="s")


def _fill2d(ref, rows, val):
    width = ref.shape[1]

    @pl.loop(0, rows)
    def _(i):
        for j in range(width // 16):
            ref[i, pl.ds(j * 16, 16)] = jnp.full((16,), val, ref.dtype)


def _compute_scatter_idx(rbuf, sidx, lo, nchunk):
    @pl.loop(0, nchunk)
    def _(ch):
        for j in range(CH // 16):
            r = rbuf[ch, pl.ds(j * 16, 16)]
            ok = (r >= lo) & (r < lo + HALF)
            sidx[ch, pl.ds(j * 16, 16)] = jnp.where(ok, r - lo, DUMMY)


@functools.cache
def _deg_kernel():
    return pl.kernel(
        _deg_body,
        out_type=jax.ShapeDtypeStruct((NPAD, DEGW), jnp.float32),
        mesh=_mesh(),
        scratch_types=[
            pltpu.VMEM((NCHUNK, CH), jnp.int32),
            pltpu.VMEM((NCHUNK, CH), jnp.int32),
            pltpu.VMEM((CH, DEGW), jnp.float32),
            pltpu.VMEM((STRIPE, DEGW), jnp.float32),
            pltpu.VMEM_SHARED((PADROWS, DEGW), jnp.float32),
        ],
    )


def _deg_body(rowp, deg_out, rbuf, sidx, ones, zstage, dacc):
    c = lax.axis_index("c")
    s = lax.axis_index("s")
    _fill2d(ones, CH, 1.0)
    _fill2d(zstage, STRIPE, 0.0)
    pltpu.sync_copy(zstage, dacc.at[pl.ds(s * STRIPE, STRIPE)])
    pltpu.sync_copy(rowp.at[pl.ds(s * NCHUNK, NCHUNK)], rbuf)
    _compute_scatter_idx(rbuf, sidx, c * HALF, NCHUNK)
    plsc.subcore_barrier()

    @pl.loop(0, NCHUNK)
    def _(ch):
        pltpu.sync_copy(ones, dacc.at[sidx.at[ch]], add=True)

    plsc.subcore_barrier()
    pltpu.sync_copy(dacc.at[pl.ds(s * STRIPE, STRIPE)],
                    deg_out.at[pl.ds(c * PADROWS + s * STRIPE, STRIPE)])


SPLIT = 2
SW = IN // SPLIT
RCH = 32
CHI = RCH * SPLIT
RNCH = EPAD // (16 * RCH)
E2ROWS = EPAD * SPLIT // CHI
ACC2 = PADROWS * SPLIT
G2 = NPAD * SPLIT
NBUF = 4
PCH = 64


@functools.cache
def _round_kernel():
    return pl.kernel(
        _round_body,
        out_type=jax.ShapeDtypeStruct((G2, SW), jnp.float32),
        mesh=_mesh(),
        scratch_types=[
            pltpu.VMEM((16,), jnp.int32),
            pltpu.VMEM((2 * NBUF, RCH), jnp.int32),
            pltpu.VMEM((2 * NBUF, CHI), jnp.int32),
            pltpu.VMEM((NBUF, RCH, SPLIT, SW), jnp.float32),
            pltpu.VMEM((PCH // SPLIT, DEGW), jnp.float32),
            pltpu.VMEM_SHARED((ACC2, SW), jnp.float32),
        ] + [pltpu.SemaphoreType.DMA] * (2 * NBUF),
        compiler_params=_sc_compiler_params(),
    )


def _round_body(g_in, cidx2, sidx2, dinv2, karr, g_out,
                kbuf, cbufg, ibufg, sbuf, dbuf, acc, *sems):
    gsem, ssem = sems[:NBUF], sems[NBUF:]
    c = lax.axis_index("c")
    s = lax.axis_index("s")
    zb = sbuf.at[0].reshape(CHI, SW)
    _fill2d(zb, CHI, 0.0)
    nz = (STRIPE * SPLIT) // CHI
    for k in range(nz):
        pltpu.sync_copy(zb,
                        acc.at[pl.ds(s * STRIPE * SPLIT + k * CHI, CHI)])
    pltpu.sync_copy(karr, kbuf)
    plsc.subcore_barrier()
    kval = lax.reduce_max(kbuf[pl.ds(0, 16)], axes=(0,))
    eblk = 2 * NBUF * RCH
    nblk_tot = EPAD // eblk
    lo = jnp.where(c == 0, 0, kval // eblk)
    hi = jnp.where(c == 0, (kval + eblk - 1) // eblk, nblk_tot)
    per = (hi - lo + 15) // 16
    bstart = lo + s * per
    bend = jnp.minimum(bstart + per, hi)

    @pl.loop(bstart, bend)
    def _(it):
        base = it * 2 * NBUF
        pltpu.sync_copy(cidx2.at[pl.ds(base, 2 * NBUF)], cbufg)
        pltpu.sync_copy(sidx2.at[pl.ds(c * E2ROWS + base, 2 * NBUF)], ibufg)
        hg = [pltpu.async_copy(g_in.at[cbufg.at[p]], sbuf.at[p], gsem[p])
              for p in range(NBUF)]
        hs = [None] * NBUF
        for p in range(NBUF):
            hg[p].wait()
            hs[p] = pltpu.async_copy(sbuf.at[p].reshape(CHI, SW),
                                     acc.at[ibufg.at[p]], ssem[p], add=True)
        for p in range(NBUF):
            hs[p].wait()
            hg[p] = pltpu.async_copy(g_in.at[cbufg.at[NBUF + p]], sbuf.at[p],
                                     gsem[p])
        for p in range(NBUF):
            hg[p].wait()
            hs[p] = pltpu.async_copy(sbuf.at[p].reshape(CHI, SW),
                                     acc.at[ibufg.at[NBUF + p]],
                                     ssem[p], add=True)
        for p in range(NBUF):
            hs[p].wait()

    plsc.subcore_barrier()
    npost = (STRIPE * SPLIT) // PCH
    for k in range(npost):
        rowbase = c * PADROWS + s * STRIPE + k * (PCH // SPLIT)
        pb = sbuf.at[0].reshape(PCH, SW)
        pltpu.sync_copy(acc.at[pl.ds(s * STRIPE * SPLIT + k * PCH, PCH)], pb)
        pltpu.sync_copy(dinv2.at[pl.ds(rowbase, PCH // SPLIT)], dbuf)

        @pl.loop(0, PCH // SPLIT)
        def _(i):
            dv = dbuf[i, pl.ds(0, 16)]
            for j in range(SPLIT):
                for l in range(SW // 16):
                    pb[i * SPLIT + j, pl.ds(l * 16, 16)] = (
                        pb[i * SPLIT + j, pl.ds(l * 16, 16)] * dv)

        pltpu.sync_copy(pb, g_out.at[pl.ds(rowbase * SPLIT, PCH)])


ROWBLK = 512
NBLK = NPAD // ROWBLK


def _prep_body(x_ref, deg_ref, g0_ref, dinv2_ref):
    xb = x_ref[...]
    db = deg_ref[...]
    fsum = jnp.sum(xb, axis=1, keepdims=True)
    finv = jnp.where(fsum != 0, 1.0 / fsum, 0.0)
    xn = xb * finv * 0.5
    d1 = db[:, 0:1]
    dinv = jnp.where(d1 > 0, lax.rsqrt(d1), 0.0)
    g0_ref[...] = xn * dinv
    dinv2_ref[...] = jnp.where(db > 0, 1.0 / db, 0.0)


_prep = pl.pallas_call(
    _prep_body,
    grid=(NBLK,),
    in_specs=[
        pl.BlockSpec((ROWBLK, IN), lambda i: (i, 0)),
        pl.BlockSpec((ROWBLK, DEGW), lambda i: (i, 0)),
    ],
    out_specs=[
        pl.BlockSpec((ROWBLK, IN), lambda i: (i, 0)),
        pl.BlockSpec((ROWBLK, DEGW), lambda i: (i, 0)),
    ],
    out_shape=[
        jax.ShapeDtypeStruct((NPAD, IN), jnp.float32),
        jax.ShapeDtypeStruct((NPAD, DEGW), jnp.float32),
    ],
)


def _mlp_body(x_ref, deg_ref, g1_ref, g2_ref, g3_ref, g4_ref,
              W1_ref, b1_ref, W2_ref, b2_ref,
              gm1_ref, bt1_ref, gm2_ref, bt2_ref, o_ref):
    xb = x_ref[...]
    fsum = jnp.sum(xb, axis=1, keepdims=True)
    finv = jnp.where(fsum != 0, 1.0 / fsum, 0.0)
    xn = xb * finv * 0.5
    sq = jnp.sqrt(deg_ref[:, 0:1])
    gacc = g1_ref[...] + g2_ref[...] + g3_ref[...] + g4_ref[...]
    y = (xn + sq * gacc) * 0.2
    a = y * (C1 * gm1_ref[...]) + bt1_ref[...]
    h = jnp.dot(a, W1_ref[...], preferred_element_type=jnp.float32) + b1_ref[...]
    h = jnp.maximum(h, 0.0)
    h = h * (C1 * gm2_ref[...]) + bt2_ref[...]
    o_ref[...] = jnp.dot(h, W2_ref[...],
                         preferred_element_type=jnp.float32) + b2_ref[...]


_mlp = pl.pallas_call(
    _mlp_body,
    grid=(NBLK,),
    in_specs=[
        pl.BlockSpec((ROWBLK, IN), lambda i: (i, 0)),
        pl.BlockSpec((ROWBLK, DEGW), lambda i: (i, 0)),
        pl.BlockSpec((ROWBLK, IN), lambda i: (i, 0)),
        pl.BlockSpec((ROWBLK, IN), lambda i: (i, 0)),
        pl.BlockSpec((ROWBLK, IN), lambda i: (i, 0)),
        pl.BlockSpec((ROWBLK, IN), lambda i: (i, 0)),
        pl.BlockSpec((IN, HID), lambda i: (0, 0)),
        pl.BlockSpec((1, HID), lambda i: (0, 0)),
        pl.BlockSpec((HID, OUT), lambda i: (0, 0)),
        pl.BlockSpec((1, OUT), lambda i: (0, 0)),
        pl.BlockSpec((1, IN), lambda i: (0, 0)),
        pl.BlockSpec((1, IN), lambda i: (0, 0)),
        pl.BlockSpec((1, HID), lambda i: (0, 0)),
        pl.BlockSpec((1, HID), lambda i: (0, 0)),
    ],
    out_specs=pl.BlockSpec((ROWBLK, OUT), lambda i: (i, 0)),
    out_shape=jax.ShapeDtypeStruct((NPAD, OUT), jnp.float32),
)


def kernel(x, edge_index, W1, b1, W2, b2, gamma1, beta1, gamma2, beta2):
    row = edge_index[0].astype(jnp.int32)
    col = edge_index[1].astype(jnp.int32)
    colr = col + jnp.where(col >= HALF, PADROWS - HALF, 0).astype(jnp.int32)
    rowflat = jnp.concatenate([row, jnp.full((EPAD - E,), SENTINEL, jnp.int32)])
    colflat = jnp.concatenate([colr, jnp.zeros((EPAD - E,), jnp.int32)])
    perm = jnp.argsort((rowflat >= HALF).astype(jnp.int8), stable=True)
    rowflat = rowflat[perm]
    colflat = colflat[perm]
    ksplit = jnp.sum(rowflat < HALF).astype(jnp.int32)
    karr = jnp.full((16,), ksplit, jnp.int32)
    rowp = rowflat.reshape(EROWS, CH)
    lanes = jnp.arange(SPLIT, dtype=jnp.int32)
    cidxr = colflat.reshape(EPAD // RCH, RCH)
    sidx_c = []
    for c in range(2):
        lo = c * HALF
        loc = jnp.where((rowflat >= lo) & (rowflat < lo + HALF),
                        rowflat - lo, DUMMY)
        sidx_c.append((loc[:, None] * SPLIT + lanes).reshape(E2ROWS, CHI))
    sidx2 = jnp.concatenate(sidx_c, axis=0)
    z = jnp.zeros((PADROWS - HALF, IN), jnp.float32)
    x_pad = jnp.concatenate([x[:HALF], z, x[HALF:], z], axis=0)

    deg = _deg_kernel()(rowp)
    g0, dinv2 = _prep(x_pad, deg)
    rnd = _round_kernel()
    r3 = lambda g: g.reshape(NPAD, SPLIT, SW)
    r256 = lambda g: g.reshape(NPAD, IN)
    g1 = rnd(r3(g0), cidxr, sidx2, dinv2, karr)
    g2 = rnd(r3(g1), cidxr, sidx2, dinv2, karr)
    g3 = rnd(r3(g2), cidxr, sidx2, dinv2, karr)
    g4 = rnd(r3(g3), cidxr, sidx2, dinv2, karr)
    g1, g2, g3, g4 = r256(g1), r256(g2), r256(g3), r256(g4)
    out_pad = _mlp(x_pad, deg, g1, g2, g3, g4,
                   W1, b1.reshape(1, HID), W2, b2.reshape(1, OUT),
                   gamma1.reshape(1, IN), beta1.reshape(1, IN),
                   gamma2.reshape(1, HID), beta2.reshape(1, HID))
    return jnp.concatenate([out_pad[:HALF], out_pad[PADROWS:PADROWS + HALF]],
                           axis=0)
